# Initial kernel scaffold; baseline (speedup 1.0000x reference)
import jax
import jax.numpy as jnp
from jax.experimental import pallas as pl


def kernel(query_key, memory_keys, memory_values):
    raise NotImplementedError("write your pallas kernel here")


# speedup vs baseline: 16.2070x; 16.2070x over previous
import functools

import jax
import jax.numpy as jnp
from jax.experimental import pallas as pl
from jax.experimental.pallas import tpu as pltpu

_TOP_K = 32
_TAU = 0.07


def _body(q_ref, k_ref, v_ref, o_ref, logits, qn, rmax, thr, den,
          *, kbs, num_kb, top_k, tau):
    ph = pl.program_id(1)
    kb = pl.program_id(2)

    @pl.when(jnp.logical_and(ph == 0, kb == 0))
    def _():
        q = q_ref[...]
        n = jnp.maximum(jnp.sqrt(jnp.sum(q * q, axis=1, keepdims=True)), 1e-12)
        qn[...] = q / n

    @pl.when(ph == 0)
    def _():
        k = k_ref[...]
        n = jnp.maximum(jnp.sqrt(jnp.sum(k * k, axis=1, keepdims=True)), 1e-12)
        kn = k / n
        logits[:, pl.ds(kb * kbs, kbs)] = jax.lax.dot_general(
            qn[...], kn, (((1,), (1,)), ((), ())),
            preferred_element_type=jnp.float32) / tau

    @pl.when(jnp.logical_and(ph == 1, kb == 0))
    def _():
        l = logits[...]
        m = jnp.max(l, axis=1, keepdims=True)
        rmax[...] = m

        def it(_, t):
            return jnp.max(jnp.where(l < t, l, -jnp.inf), axis=1,
                           keepdims=True)

        t = jax.lax.fori_loop(0, top_k - 1, it, m)
        thr[...] = t
        w = jnp.where(l >= t, jnp.exp(l - m), 0.0)
        logits[...] = w
        den[...] = jnp.sum(w, axis=1, keepdims=True)

    @pl.when(ph == 1)
    def _():
        p = logits[:, pl.ds(kb * kbs, kbs)]
        acc = jax.lax.dot_general(
            p, v_ref[...], (((1,), (0,)), ((), ())),
            preferred_element_type=jnp.float32,
            precision=jax.lax.Precision.HIGHEST)

        @pl.when(kb == 0)
        def _():
            o_ref[...] = acc

        @pl.when(kb > 0)
        def _():
            o_ref[...] = o_ref[...] + acc

    @pl.when(jnp.logical_and(ph == 1, kb == num_kb - 1))
    def _():
        o_ref[...] = o_ref[...] / den[...]


def _memory_read_2d(q2, k2, v2, *, qbs, kbs):
    nq, ck = q2.shape
    nm, cv = v2.shape
    num_qb = nq // qbs
    num_kb = nm // kbs

    body = functools.partial(_body, kbs=kbs, num_kb=num_kb,
                             top_k=_TOP_K, tau=_TAU)
    return pl.pallas_call(
        body,
        grid=(num_qb, 2, num_kb),
        in_specs=[
            pl.BlockSpec((qbs, ck), lambda qb, ph, kb: (qb, 0)),
            pl.BlockSpec((kbs, ck),
                         lambda qb, ph, kb: (jnp.where(ph == 0, kb, 0), 0)),
            pl.BlockSpec((kbs, cv),
                         lambda qb, ph, kb: (jnp.where(ph == 1, kb, 0), 0)),
        ],
        out_specs=pl.BlockSpec((qbs, cv), lambda qb, ph, kb: (qb, 0)),
        out_shape=jax.ShapeDtypeStruct((nq, cv), jnp.float32),
        scratch_shapes=[
            pltpu.VMEM((qbs, nm), jnp.float32),
            pltpu.VMEM((qbs, ck), jnp.float32),
            pltpu.VMEM((qbs, 1), jnp.float32),
            pltpu.VMEM((qbs, 1), jnp.float32),
            pltpu.VMEM((qbs, 1), jnp.float32),
        ],
    )(q2, k2, v2)


def kernel(query_key, memory_keys, memory_values):
    b, ck, hq, wq = query_key.shape
    _, cv, hm, wm = memory_values.shape
    nq = hq * wq
    nm = hm * wm
    q2 = query_key.reshape(ck, nq).T
    k2 = memory_keys.reshape(ck, nm).T
    v2 = memory_values.reshape(cv, nm).T
    qbs = min(128, nq)
    kbs = min(4096, nm)
    read = _memory_read_2d(q2, k2, v2, qbs=qbs, kbs=kbs)
    return read.T.reshape(b, cv, hq, wq)


# speedup vs baseline: 31.9758x; 1.9730x over previous
import functools

import jax
import jax.numpy as jnp
from jax.experimental import pallas as pl
from jax.experimental.pallas import tpu as pltpu

_TOP_K = 32
_TAU = 0.07


def _norm_rows_body(x_ref, o_ref):
    x = x_ref[...]
    n = jnp.maximum(jnp.sqrt(jnp.sum(x * x, axis=1, keepdims=True)), 1e-12)
    o_ref[...] = x / n


def _normalize_rows(x, bs):
    n, c = x.shape
    return pl.pallas_call(
        _norm_rows_body,
        grid=(n // bs,),
        in_specs=[pl.BlockSpec((bs, c), lambda i: (i, 0))],
        out_specs=pl.BlockSpec((bs, c), lambda i: (i, 0)),
        out_shape=jax.ShapeDtypeStruct((n, c), jnp.float32),
    )(x)


def _body(q_ref, k_ref, v_ref, o_ref, logits, qn, rmax, thr, den,
          *, qbs, kbs, num_kb, nm, top_k, tau):
    ph = pl.program_id(1)
    kb = pl.program_id(2)

    @pl.when(jnp.logical_and(ph == 0, kb == 0))
    def _():
        q = q_ref[...]
        n = jnp.maximum(jnp.sqrt(jnp.sum(q * q, axis=1, keepdims=True)), 1e-12)
        qn[...] = q / n

    @pl.when(ph == 0)
    def _():
        logits[:, pl.ds(kb * kbs, kbs)] = jax.lax.dot_general(
            qn[...], k_ref[...], (((1,), (1,)), ((), ())),
            preferred_element_type=jnp.float32) / tau

    @pl.when(jnp.logical_and(ph == 1, kb == 0))
    def _():
        l = logits[...]
        if nm % 1024 == 0:
            cm = jnp.max(l.reshape(qbs, nm // 1024, 8, 128), axis=2)
            cm = cm.reshape(qbs, nm // 8)
        else:
            cm = l
        m = jnp.max(cm, axis=1, keepdims=True)
        rmax[...] = m

        def it(_, t):
            return jnp.max(jnp.where(cm < t, cm, -jnp.inf), axis=1,
                           keepdims=True)

        c32 = jax.lax.fori_loop(0, top_k - 1, it, m)
        cnt = jnp.sum(jnp.where(l >= c32, 1.0, 0.0), axis=1, keepdims=True)
        n_extra = jnp.max(cnt) - float(top_k)
        n_extra_i = n_extra.astype(jnp.int32)

        def fix(_, carry):
            t, n = carry
            live = n > float(top_k)
            mn = jnp.min(jnp.where(l >= t, l, jnp.inf), axis=1, keepdims=True)
            t2 = jnp.min(jnp.where(l > mn, l, jnp.inf), axis=1, keepdims=True)
            t = jnp.where(live, t2, t)
            n = jnp.where(live, n - 1.0, n)
            return t, n

        t, _ = jax.lax.fori_loop(0, n_extra_i, fix,
                                 (jnp.broadcast_to(c32, (qbs, 1)), cnt))
        thr[...] = t
        w = jnp.where(l >= t, jnp.exp(l - m), 0.0)
        logits[...] = w
        den[...] = jnp.sum(w, axis=1, keepdims=True)

    @pl.when(ph == 1)
    def _():
        p = logits[:, pl.ds(kb * kbs, kbs)]
        acc = jax.lax.dot_general(
            p, v_ref[...], (((1,), (0,)), ((), ())),
            preferred_element_type=jnp.float32)

        @pl.when(kb == 0)
        def _():
            o_ref[...] = acc

        @pl.when(kb > 0)
        def _():
            o_ref[...] = o_ref[...] + acc

    @pl.when(jnp.logical_and(ph == 1, kb == num_kb - 1))
    def _():
        o_ref[...] = o_ref[...] / den[...]


def _memory_read_2d(q2, kn2, v2, *, qbs, kbs):
    nq, ck = q2.shape
    nm, cv = v2.shape
    num_qb = nq // qbs
    num_kb = nm // kbs

    body = functools.partial(_body, qbs=qbs, kbs=kbs, num_kb=num_kb, nm=nm,
                             top_k=_TOP_K, tau=_TAU)
    return pl.pallas_call(
        body,
        grid=(num_qb, 2, num_kb),
        in_specs=[
            pl.BlockSpec((qbs, ck), lambda qb, ph, kb: (qb, 0)),
            pl.BlockSpec((kbs, ck),
                         lambda qb, ph, kb: (jnp.where(ph == 0, kb, 0), 0)),
            pl.BlockSpec((kbs, cv),
                         lambda qb, ph, kb: (jnp.where(ph == 1, kb, 0), 0)),
        ],
        out_specs=pl.BlockSpec((qbs, cv), lambda qb, ph, kb: (qb, 0)),
        out_shape=jax.ShapeDtypeStruct((nq, cv), jnp.float32),
        scratch_shapes=[
            pltpu.VMEM((qbs, nm), jnp.float32),
            pltpu.VMEM((qbs, ck), jnp.float32),
            pltpu.VMEM((qbs, 1), jnp.float32),
            pltpu.VMEM((qbs, 1), jnp.float32),
            pltpu.VMEM((qbs, 1), jnp.float32),
        ],
    )(q2, kn2, v2)


def kernel(query_key, memory_keys, memory_values):
    b, ck, hq, wq = query_key.shape
    _, cv, hm, wm = memory_values.shape
    nq = hq * wq
    nm = hm * wm
    q2 = query_key.reshape(ck, nq).T
    k2 = memory_keys.reshape(ck, nm).T
    v2 = memory_values.reshape(cv, nm).T
    kn2 = _normalize_rows(k2, min(4096, nm))
    qbs = min(128, nq)
    kbs = min(4096, nm)
    read = _memory_read_2d(q2, kn2, v2, qbs=qbs, kbs=kbs)
    return read.T.reshape(b, cv, hq, wq)


# speedup vs baseline: 35.2893x; 1.1036x over previous
import functools

import jax
import jax.numpy as jnp
from jax.experimental import pallas as pl
from jax.experimental.pallas import tpu as pltpu

_TOP_K = 32
_TAU = 0.07


def _norm_rows_body(x_ref, o_ref):
    x = x_ref[...]
    n = jnp.maximum(jnp.sqrt(jnp.sum(x * x, axis=1, keepdims=True)), 1e-12)
    o_ref[...] = x / n


def _normalize_rows(x, bs):
    n, c = x.shape
    return pl.pallas_call(
        _norm_rows_body,
        grid=(n // bs,),
        in_specs=[pl.BlockSpec((bs, c), lambda i: (i, 0))],
        out_specs=pl.BlockSpec((bs, c), lambda i: (i, 0)),
        out_shape=jax.ShapeDtypeStruct((n, c), jnp.float32),
    )(x)


def _body(q_ref, k_ref, v_ref, o_ref, logits, cms, qn, rmax, thr, den,
          *, qbs, kbs, num_kb, nm, top_k, tau, chunked):
    ph = pl.program_id(1)
    kb = pl.program_id(2)

    @pl.when(jnp.logical_and(ph == 0, kb == 0))
    def _():
        q = q_ref[...]
        n = jnp.maximum(jnp.sqrt(jnp.sum(q * q, axis=1, keepdims=True)), 1e-12)
        qn[...] = q / n

    @pl.when(ph == 0)
    def _():
        lb = jax.lax.dot_general(
            qn[...], k_ref[...], (((1,), (1,)), ((), ())),
            preferred_element_type=jnp.float32) / tau
        logits[:, pl.ds(kb * kbs, kbs)] = lb
        if chunked:
            cmb = jnp.max(lb.reshape(qbs, kbs // 1024, 8, 128), axis=2)
            cms[:, pl.ds(kb * (kbs // 8), kbs // 8)] = cmb.reshape(
                qbs, kbs // 8)
        else:
            cms[:, pl.ds(kb * kbs, kbs)] = lb

    @pl.when(jnp.logical_and(ph == 1, kb == 0))
    def _():
        l = logits[...]
        cm = cms[...]
        m = jnp.max(cm, axis=1, keepdims=True)
        rmax[...] = m

        def it(_, t):
            return jnp.max(jnp.where(cm < t, cm, -jnp.inf), axis=1,
                           keepdims=True)

        c32 = jax.lax.fori_loop(0, top_k - 1, it, m)
        cnt = jnp.sum(jnp.where(l >= c32, 1.0, 0.0), axis=1, keepdims=True)
        n_it = (jnp.max(cnt) - float(top_k - 1)).astype(jnp.int32)

        def fix(_, carry):
            t, r = carry
            live = r > 0.0
            cand = jnp.logical_and(l >= c32, l > t)
            t2 = jnp.min(jnp.where(cand, l, jnp.inf), axis=1, keepdims=True)
            t = jnp.where(live, t2, t)
            r = r - jnp.where(live, 1.0, 0.0)
            return t, r

        t, _ = jax.lax.fori_loop(
            0, n_it, fix,
            (jnp.full((qbs, 1), -jnp.inf, jnp.float32),
             cnt - float(top_k - 1)))
        thr[...] = t

    @pl.when(ph == 1)
    def _():
        lb = logits[:, pl.ds(kb * kbs, kbs)]
        w = jnp.where(lb >= thr[...], jnp.exp(lb - rmax[...]), 0.0)
        acc = jax.lax.dot_general(
            w, v_ref[...], (((1,), (0,)), ((), ())),
            preferred_element_type=jnp.float32)
        dw = jnp.sum(w, axis=1, keepdims=True)

        @pl.when(kb == 0)
        def _():
            o_ref[...] = acc
            den[...] = dw

        @pl.when(kb > 0)
        def _():
            o_ref[...] = o_ref[...] + acc
            den[...] = den[...] + dw

    @pl.when(jnp.logical_and(ph == 1, kb == num_kb - 1))
    def _():
        o_ref[...] = o_ref[...] / den[...]


def _memory_read_2d(q2, kn2, v2, *, qbs, kbs):
    nq, ck = q2.shape
    nm, cv = v2.shape
    num_qb = nq // qbs
    num_kb = nm // kbs
    chunked = kbs % 1024 == 0
    cm_w = nm // 8 if chunked else nm

    body = functools.partial(_body, qbs=qbs, kbs=kbs, num_kb=num_kb, nm=nm,
                             top_k=_TOP_K, tau=_TAU, chunked=chunked)
    return pl.pallas_call(
        body,
        grid=(num_qb, 2, num_kb),
        in_specs=[
            pl.BlockSpec((qbs, ck), lambda qb, ph, kb: (qb, 0)),
            pl.BlockSpec((kbs, ck),
                         lambda qb, ph, kb: (jnp.where(ph == 0, kb, 0), 0)),
            pl.BlockSpec((kbs, cv),
                         lambda qb, ph, kb: (jnp.where(ph == 1, kb, 0), 0)),
        ],
        out_specs=pl.BlockSpec((qbs, cv), lambda qb, ph, kb: (qb, 0)),
        out_shape=jax.ShapeDtypeStruct((nq, cv), jnp.float32),
        scratch_shapes=[
            pltpu.VMEM((qbs, nm), jnp.float32),
            pltpu.VMEM((qbs, cm_w), jnp.float32),
            pltpu.VMEM((qbs, ck), jnp.float32),
            pltpu.VMEM((qbs, 1), jnp.float32),
            pltpu.VMEM((qbs, 1), jnp.float32),
            pltpu.VMEM((qbs, 1), jnp.float32),
        ],
    )(q2, kn2, v2)


def kernel(query_key, memory_keys, memory_values):
    b, ck, hq, wq = query_key.shape
    _, cv, hm, wm = memory_values.shape
    nq = hq * wq
    nm = hm * wm
    q2 = query_key.reshape(ck, nq).T
    k2 = memory_keys.reshape(ck, nm).T
    v2 = memory_values.reshape(cv, nm).T
    kn2 = _normalize_rows(k2, min(4096, nm))
    qbs = min(128, nq)
    kbs = min(4096, nm)
    read = _memory_read_2d(q2, kn2, v2, qbs=qbs, kbs=kbs)
    return read.T.reshape(b, cv, hq, wq)


# speedup vs baseline: 38.0601x; 1.0785x over previous
import functools

import jax
import jax.numpy as jnp
from jax.experimental import pallas as pl
from jax.experimental.pallas import tpu as pltpu

_TOP_K = 32
_TAU = 0.07


def _norm_rows_body(x_ref, o_ref):
    x = x_ref[...]
    n = jnp.maximum(jnp.sqrt(jnp.sum(x * x, axis=1, keepdims=True)), 1e-12)
    o_ref[...] = x / n


def _normalize_rows(x, bs):
    n, c = x.shape
    return pl.pallas_call(
        _norm_rows_body,
        grid=(n // bs,),
        in_specs=[pl.BlockSpec((bs, c), lambda i: (i, 0))],
        out_specs=pl.BlockSpec((bs, c), lambda i: (i, 0)),
        out_shape=jax.ShapeDtypeStruct((n, c), jnp.float32),
    )(x)


def _body(q_ref, k_ref, v_ref, o_ref, logits, cms, qn, rmax, thr, den,
          *, qbs, kbs, num_kb, nm, top_k, tau, chunked):
    ph = pl.program_id(1)
    kb = pl.program_id(2)

    @pl.when(jnp.logical_and(ph == 0, kb == 0))
    def _():
        q = q_ref[...]
        n = jnp.maximum(jnp.sqrt(jnp.sum(q * q, axis=1, keepdims=True)), 1e-12)
        qn[...] = q / n

    @pl.when(ph == 0)
    def _():
        lb = jax.lax.dot_general(
            qn[...], k_ref[...], (((1,), (1,)), ((), ())),
            preferred_element_type=jnp.float32) / tau
        logits[:, pl.ds(kb * kbs, kbs)] = lb
        if chunked:
            cmb = jnp.max(lb.reshape(qbs, kbs // 1024, 8, 128), axis=2)
            cms[:, pl.ds(kb * (kbs // 8), kbs // 8)] = cmb.reshape(
                qbs, kbs // 8)
        else:
            cms[:, pl.ds(kb * kbs, kbs)] = lb

    @pl.when(jnp.logical_and(ph == 1, kb == 0))
    def _():
        l = logits[...]
        cm = cms[...]
        m = jnp.max(cm, axis=1, keepdims=True)
        rmax[...] = m

        def kth_largest(arr, init, k):
            def it(_, t):
                return jnp.max(jnp.where(arr < t, arr, -jnp.inf), axis=1,
                               keepdims=True)

            return jax.lax.fori_loop(0, k - 1, it, init)

        def refine(arr, c_lo, k):
            cnt = jnp.sum(jnp.where(arr >= c_lo, 1.0, 0.0), axis=1,
                          keepdims=True)
            n_it = (jnp.max(cnt) - float(k - 1)).astype(jnp.int32)

            def fix(_, carry):
                t, r = carry
                live = r > 0.0
                cand = jnp.logical_and(arr >= c_lo, arr > t)
                t2 = jnp.min(jnp.where(cand, arr, jnp.inf), axis=1,
                             keepdims=True)
                t = jnp.where(live, t2, t)
                r = r - jnp.where(live, 1.0, 0.0)
                return t, r

            t, _ = jax.lax.fori_loop(
                0, n_it, fix,
                (jnp.full((arr.shape[0], 1), -jnp.inf, jnp.float32),
                 cnt - float(k - 1)))
            return t

        cmw = cm.shape[1]
        if chunked and cmw % 1024 == 0:
            cm2 = jnp.max(cm.reshape(qbs, cmw // 1024, 8, 128),
                          axis=2).reshape(qbs, cmw // 8)
            c2 = kth_largest(cm2, m, top_k)
            c32 = refine(cm, c2, top_k)
        else:
            c32 = kth_largest(cm, m, top_k)
        t = refine(l, c32, top_k)
        thr[...] = t

    @pl.when(ph == 1)
    def _():
        lb = logits[:, pl.ds(kb * kbs, kbs)]
        w = jnp.where(lb >= thr[...], jnp.exp(lb - rmax[...]), 0.0)
        acc = jax.lax.dot_general(
            w, v_ref[...], (((1,), (0,)), ((), ())),
            preferred_element_type=jnp.float32)
        dw = jnp.sum(w, axis=1, keepdims=True)

        @pl.when(kb == 0)
        def _():
            o_ref[...] = acc
            den[...] = dw

        @pl.when(kb > 0)
        def _():
            o_ref[...] = o_ref[...] + acc
            den[...] = den[...] + dw

    @pl.when(jnp.logical_and(ph == 1, kb == num_kb - 1))
    def _():
        o_ref[...] = o_ref[...] / den[...]


def _memory_read_2d(q2, kn2, v2, *, qbs, kbs):
    nq, ck = q2.shape
    nm, cv = v2.shape
    num_qb = nq // qbs
    num_kb = nm // kbs
    chunked = kbs % 1024 == 0
    cm_w = nm // 8 if chunked else nm

    body = functools.partial(_body, qbs=qbs, kbs=kbs, num_kb=num_kb, nm=nm,
                             top_k=_TOP_K, tau=_TAU, chunked=chunked)
    return pl.pallas_call(
        body,
        grid=(num_qb, 2, num_kb),
        in_specs=[
            pl.BlockSpec((qbs, ck), lambda qb, ph, kb: (qb, 0)),
            pl.BlockSpec((kbs, ck),
                         lambda qb, ph, kb: (jnp.where(ph == 0, kb, 0), 0)),
            pl.BlockSpec((kbs, cv),
                         lambda qb, ph, kb: (jnp.where(ph == 1, kb, 0), 0)),
        ],
        out_specs=pl.BlockSpec((qbs, cv), lambda qb, ph, kb: (qb, 0)),
        out_shape=jax.ShapeDtypeStruct((nq, cv), jnp.float32),
        scratch_shapes=[
            pltpu.VMEM((qbs, nm), jnp.float32),
            pltpu.VMEM((qbs, cm_w), jnp.float32),
            pltpu.VMEM((qbs, ck), jnp.float32),
            pltpu.VMEM((qbs, 1), jnp.float32),
            pltpu.VMEM((qbs, 1), jnp.float32),
            pltpu.VMEM((qbs, 1), jnp.float32),
        ],
    )(q2, kn2, v2)


def kernel(query_key, memory_keys, memory_values):
    b, ck, hq, wq = query_key.shape
    _, cv, hm, wm = memory_values.shape
    nq = hq * wq
    nm = hm * wm
    q2 = query_key.reshape(ck, nq).T
    k2 = memory_keys.reshape(ck, nm).T
    v2 = memory_values.reshape(cv, nm).T
    kn2 = _normalize_rows(k2, min(4096, nm))
    qbs = min(128, nq)
    kbs = min(4096, nm)
    read = _memory_read_2d(q2, kn2, v2, qbs=qbs, kbs=kbs)
    return read.T.reshape(b, cv, hq, wq)


# speedup vs baseline: 41.9735x; 1.1028x over previous
import functools

import jax
import jax.numpy as jnp
from jax.experimental import pallas as pl
from jax.experimental.pallas import tpu as pltpu

_TOP_K = 32
_TAU = 0.07


def _norm_rows_body(x_ref, o_ref):
    x = x_ref[...]
    n = jnp.maximum(jnp.sqrt(jnp.sum(x * x, axis=1, keepdims=True)), 1e-12)
    o_ref[...] = x / n


def _normalize_rows(x, bs):
    n, c = x.shape
    return pl.pallas_call(
        _norm_rows_body,
        grid=(n // bs,),
        in_specs=[pl.BlockSpec((bs, c), lambda i: (i, 0))],
        out_specs=pl.BlockSpec((bs, c), lambda i: (i, 0)),
        out_shape=jax.ShapeDtypeStruct((n, c), jnp.float32),
    )(x)


def _body(q_ref, k_ref, v_ref, o_ref, logits, cms, qn, rmax, thr, den,
          *, qbs, kbs, num_kb, nm, top_k, tau, chunked):
    ph = pl.program_id(1)
    kb = pl.program_id(2)

    @pl.when(jnp.logical_and(ph == 0, kb == 0))
    def _():
        q = q_ref[...]
        n = jnp.maximum(jnp.sqrt(jnp.sum(q * q, axis=1, keepdims=True)), 1e-12)
        qn[...] = q / n

    @pl.when(ph == 0)
    def _():
        lb = jax.lax.dot_general(
            qn[...], k_ref[...], (((1,), (1,)), ((), ())),
            preferred_element_type=jnp.float32) / tau
        logits[:, pl.ds(kb * kbs, kbs)] = lb
        if chunked:
            cmb = jnp.max(lb.reshape(qbs, kbs // 1024, 8, 128), axis=2)
            cms[:, pl.ds(kb * (kbs // 8), kbs // 8)] = cmb.reshape(
                qbs, kbs // 8)
        else:
            cms[:, pl.ds(kb * kbs, kbs)] = lb

    @pl.when(jnp.logical_and(ph == 1, kb == 0))
    def _():
        l = logits[...]
        cm = cms[...]
        m = jnp.max(cm, axis=1, keepdims=True)
        rmax[...] = m

        def kth_largest(arr, init, k):
            def it(_, t):
                return jnp.max(jnp.where(arr < t, arr, -jnp.inf), axis=1,
                               keepdims=True)

            return jax.lax.fori_loop(0, k - 1, it, init)

        def refine(arr, c_lo, k):
            cnt = jnp.sum(jnp.where(arr >= c_lo, 1.0, 0.0), axis=1,
                          keepdims=True)
            n_it = (jnp.max(cnt) - float(k - 1)).astype(jnp.int32)

            def fix(_, carry):
                t, r = carry
                live = r > 0.0
                cand = jnp.logical_and(arr >= c_lo, arr > t)
                t2 = jnp.min(jnp.where(cand, arr, jnp.inf), axis=1,
                             keepdims=True)
                t = jnp.where(live, t2, t)
                r = r - jnp.where(live, 1.0, 0.0)
                return t, r

            t, _ = jax.lax.fori_loop(
                0, n_it, fix,
                (jnp.full((arr.shape[0], 1), -jnp.inf, jnp.float32),
                 cnt - float(k - 1)))
            return t

        cmw = cm.shape[1]
        if chunked and cmw % 1024 == 0:
            cm2 = jnp.max(cm.reshape(qbs, cmw // 1024, 8, 128),
                          axis=2).reshape(qbs, cmw // 8)
            c2 = kth_largest(cm2, m, top_k)
            c32 = refine(cm, c2, top_k)
        else:
            c32 = kth_largest(cm, m, top_k)
        t = refine(l, c32, top_k)
        thr[...] = t

    @pl.when(ph == 1)
    def _():
        lb = logits[:, pl.ds(kb * kbs, kbs)]
        w = jnp.where(lb >= thr[...], jnp.exp(lb - rmax[...]), 0.0)
        acc = jax.lax.dot_general(
            w.astype(jnp.bfloat16), v_ref[...], (((1,), (0,)), ((), ())),
            preferred_element_type=jnp.float32)
        dw = jnp.sum(w, axis=1, keepdims=True)

        @pl.when(kb == 0)
        def _():
            o_ref[...] = acc
            den[...] = dw

        @pl.when(kb > 0)
        def _():
            o_ref[...] = o_ref[...] + acc
            den[...] = den[...] + dw

    @pl.when(jnp.logical_and(ph == 1, kb == num_kb - 1))
    def _():
        o_ref[...] = o_ref[...] / den[...]


def _memory_read_2d(q2, kn2, v2, *, qbs, kbs):
    nq, ck = q2.shape
    nm, cv = v2.shape
    num_qb = nq // qbs
    num_kb = nm // kbs
    chunked = kbs % 1024 == 0
    cm_w = nm // 8 if chunked else nm

    body = functools.partial(_body, qbs=qbs, kbs=kbs, num_kb=num_kb, nm=nm,
                             top_k=_TOP_K, tau=_TAU, chunked=chunked)
    return pl.pallas_call(
        body,
        grid=(num_qb, 2, num_kb),
        in_specs=[
            pl.BlockSpec((qbs, ck), lambda qb, ph, kb: (qb, 0)),
            pl.BlockSpec((kbs, ck),
                         lambda qb, ph, kb: (jnp.where(ph == 0, kb, 0), 0)),
            pl.BlockSpec((kbs, cv),
                         lambda qb, ph, kb: (jnp.where(ph == 1, kb, 0), 0)),
        ],
        out_specs=pl.BlockSpec((qbs, cv), lambda qb, ph, kb: (qb, 0)),
        out_shape=jax.ShapeDtypeStruct((nq, cv), jnp.float32),
        scratch_shapes=[
            pltpu.VMEM((qbs, nm), jnp.float32),
            pltpu.VMEM((qbs, cm_w), jnp.float32),
            pltpu.VMEM((qbs, ck), jnp.float32),
            pltpu.VMEM((qbs, 1), jnp.float32),
            pltpu.VMEM((qbs, 1), jnp.float32),
            pltpu.VMEM((qbs, 1), jnp.float32),
        ],
    )(q2, kn2, v2)


def kernel(query_key, memory_keys, memory_values):
    b, ck, hq, wq = query_key.shape
    _, cv, hm, wm = memory_values.shape
    nq = hq * wq
    nm = hm * wm
    q2 = query_key.reshape(ck, nq).T
    k2 = memory_keys.reshape(ck, nm).T
    v2 = memory_values.reshape(cv, nm).T.astype(jnp.bfloat16)
    kn2 = _normalize_rows(k2, min(4096, nm))
    qbs = min(128, nq)
    kbs = min(4096, nm)
    read = _memory_read_2d(q2, kn2, v2, qbs=qbs, kbs=kbs)
    return read.T.reshape(b, cv, hq, wq)


# speedup vs baseline: 47.8282x; 1.1395x over previous
import functools

import jax
import jax.numpy as jnp
from jax.experimental import pallas as pl
from jax.experimental.pallas import tpu as pltpu

_TOP_K = 32
_TAU = 0.07


def _norm_rows_body(x_ref, o_ref):
    x = x_ref[...]
    n = jnp.maximum(jnp.sqrt(jnp.sum(x * x, axis=1, keepdims=True)), 1e-12)
    o_ref[...] = x / n


def _normalize_rows(x, bs):
    n, c = x.shape
    return pl.pallas_call(
        _norm_rows_body,
        grid=(n // bs,),
        in_specs=[pl.BlockSpec((bs, c), lambda i: (i, 0))],
        out_specs=pl.BlockSpec((bs, c), lambda i: (i, 0)),
        out_shape=jax.ShapeDtypeStruct((n, c), jnp.float32),
    )(x)


def _body(q_ref, k_ref, v_ref, o_ref, logits, cms, qn, rmax, thr, den,
          *, qbs, kbs, num_kb, nm, top_k, tau, chunked):
    ph = pl.program_id(1)
    kb = pl.program_id(2)

    @pl.when(jnp.logical_and(ph == 0, kb == 0))
    def _():
        q = q_ref[...]
        n = jnp.maximum(jnp.sqrt(jnp.sum(q * q, axis=1, keepdims=True)), 1e-12)
        qn[...] = q / n

    @pl.when(ph == 0)
    def _():
        lb = jax.lax.dot_general(
            qn[...], k_ref[...], (((1,), (1,)), ((), ())),
            preferred_element_type=jnp.float32) / tau
        logits[:, pl.ds(kb * kbs, kbs)] = lb
        if chunked:
            cmb = jnp.max(lb.reshape(qbs, kbs // 2048, 16, 128), axis=2)
            cms[:, pl.ds(kb * (kbs // 16), kbs // 16)] = cmb.reshape(
                qbs, kbs // 16)
        else:
            cms[:, pl.ds(kb * kbs, kbs)] = lb

    @pl.when(jnp.logical_and(ph == 1, kb == 0))
    def _():
        l = logits[...]
        cm = cms[...]
        m = jnp.max(cm, axis=1, keepdims=True)
        rmax[...] = m

        def kth_largest(arr, init, k):
            def it(_, t):
                return jnp.max(jnp.where(arr < t, arr, -jnp.inf), axis=1,
                               keepdims=True)

            return jax.lax.fori_loop(0, k - 1, it, init)

        def refine(arr, c_lo, k):
            cnt = jnp.sum(jnp.where(arr >= c_lo, 1.0, 0.0), axis=1,
                          keepdims=True)
            n_it = (jnp.max(cnt) - float(k - 1)).astype(jnp.int32)

            def fix(_, carry):
                t, r = carry
                live = r > 0.0
                cand = jnp.logical_and(arr >= c_lo, arr > t)
                t2 = jnp.min(jnp.where(cand, arr, jnp.inf), axis=1,
                             keepdims=True)
                t = jnp.where(live, t2, t)
                r = r - jnp.where(live, 1.0, 0.0)
                return t, r

            t, _ = jax.lax.fori_loop(
                0, n_it, fix,
                (jnp.full((arr.shape[0], 1), -jnp.inf, jnp.float32),
                 cnt - float(k - 1)))
            return t

        cmw = cm.shape[1]
        if chunked and cmw % 1024 == 0:
            cm2 = jnp.max(cm.reshape(qbs, cmw // 1024, 8, 128),
                          axis=2).reshape(qbs, cmw // 8)
            c2 = kth_largest(cm2, m, top_k)
            c32 = refine(cm, c2, top_k)
        else:
            c32 = kth_largest(cm, m, top_k)
        t = refine(l, c32, top_k)
        thr[...] = t

    @pl.when(ph == 1)
    def _():
        lb = logits[:, pl.ds(kb * kbs, kbs)]
        w = jnp.where(lb >= thr[...], jnp.exp(lb - rmax[...]), 0.0)
        acc = jax.lax.dot_general(
            w.astype(jnp.bfloat16), v_ref[...], (((1,), (0,)), ((), ())),
            preferred_element_type=jnp.float32)
        dw = jnp.sum(w, axis=1, keepdims=True)

        @pl.when(kb == 0)
        def _():
            o_ref[...] = acc
            den[...] = dw

        @pl.when(kb > 0)
        def _():
            o_ref[...] = o_ref[...] + acc
            den[...] = den[...] + dw

    @pl.when(jnp.logical_and(ph == 1, kb == num_kb - 1))
    def _():
        o_ref[...] = o_ref[...] / den[...]


def _memory_read_2d(q2, kn2, v2, *, qbs, kbs):
    nq, ck = q2.shape
    nm, cv = v2.shape
    num_qb = nq // qbs
    num_kb = nm // kbs
    chunked = kbs % 2048 == 0
    cm_w = nm // 16 if chunked else nm

    body = functools.partial(_body, qbs=qbs, kbs=kbs, num_kb=num_kb, nm=nm,
                             top_k=_TOP_K, tau=_TAU, chunked=chunked)
    return pl.pallas_call(
        body,
        grid=(num_qb, 2, num_kb),
        in_specs=[
            pl.BlockSpec((qbs, ck), lambda qb, ph, kb: (qb, 0)),
            pl.BlockSpec((kbs, ck),
                         lambda qb, ph, kb: (jnp.where(ph == 0, kb, 0), 0)),
            pl.BlockSpec((kbs, cv),
                         lambda qb, ph, kb: (jnp.where(ph == 1, kb, 0), 0)),
        ],
        out_specs=pl.BlockSpec((qbs, cv), lambda qb, ph, kb: (qb, 0)),
        out_shape=jax.ShapeDtypeStruct((nq, cv), jnp.float32),
        scratch_shapes=[
            pltpu.VMEM((qbs, nm), jnp.float32),
            pltpu.VMEM((qbs, cm_w), jnp.float32),
            pltpu.VMEM((qbs, ck), jnp.float32),
            pltpu.VMEM((qbs, 1), jnp.float32),
            pltpu.VMEM((qbs, 1), jnp.float32),
            pltpu.VMEM((qbs, 1), jnp.float32),
        ],
    )(q2, kn2, v2)


def kernel(query_key, memory_keys, memory_values):
    b, ck, hq, wq = query_key.shape
    _, cv, hm, wm = memory_values.shape
    nq = hq * wq
    nm = hm * wm
    q2 = query_key.reshape(ck, nq).T
    k2 = memory_keys.reshape(ck, nm).T
    v2 = memory_values.reshape(cv, nm).T.astype(jnp.bfloat16)
    kn2 = _normalize_rows(k2, min(4096, nm))
    qbs = min(128, nq)
    kbs = min(8192, nm)
    read = _memory_read_2d(q2, kn2, v2, qbs=qbs, kbs=kbs)
    return read.T.reshape(b, cv, hq, wq)


# speedup vs baseline: 48.1654x; 1.0070x over previous
import functools

import jax
import jax.numpy as jnp
from jax.experimental import pallas as pl
from jax.experimental.pallas import tpu as pltpu

_TOP_K = 32
_TAU = 0.07


def _norm_rows_body(x_ref, o_ref):
    x = x_ref[...]
    n = jnp.maximum(jnp.sqrt(jnp.sum(x * x, axis=1, keepdims=True)), 1e-12)
    o_ref[...] = x / n


def _normalize_rows(x, bs):
    n, c = x.shape
    return pl.pallas_call(
        _norm_rows_body,
        grid=(n // bs,),
        in_specs=[pl.BlockSpec((bs, c), lambda i: (i, 0))],
        out_specs=pl.BlockSpec((bs, c), lambda i: (i, 0)),
        out_shape=jax.ShapeDtypeStruct((n, c), jnp.float32),
    )(x)


def _body(q_ref, k_ref, v_ref, o_ref, logits, cms, scal,
          *, qbs, kbs, num_kb, nm, top_k, tau, chunked):
    ph = pl.program_id(1)
    kb = pl.program_id(2)

    @pl.when(ph == 0)
    def _():
        q = q_ref[...]
        qn = q / jnp.maximum(
            jnp.sqrt(jnp.sum(q * q, axis=1, keepdims=True)), 1e-12)
        lb = jax.lax.dot_general(
            qn, k_ref[...], (((1,), (1,)), ((), ())),
            preferred_element_type=jnp.float32) / tau
        logits[:, pl.ds(kb * kbs, kbs)] = lb
        if chunked:
            cmb = jnp.max(lb.reshape(qbs, kbs // 1024, 8, 128), axis=2)
            cms[:, pl.ds(kb * (kbs // 8), kbs // 8)] = cmb.reshape(
                qbs, kbs // 8)
        else:
            cms[:, pl.ds(kb * kbs, kbs)] = lb

    @pl.when(jnp.logical_and(ph == 1, kb == 0))
    def _():
        l = logits[...]
        cm = cms[...]
        m = jnp.max(cm, axis=1, keepdims=True)
        scal[:, 0:1] = m

        def kth_largest(arr, init, k):
            def it(_, t):
                return jnp.max(jnp.where(arr < t, arr, -jnp.inf), axis=1,
                               keepdims=True)

            return jax.lax.fori_loop(0, k - 1, it, init)

        def refine(arr, c_lo, k):
            masked = jnp.where(arr >= c_lo, arr, jnp.inf)
            cnt = jnp.sum(jnp.where(arr >= c_lo, 1.0, 0.0), axis=1,
                          keepdims=True)
            t0 = jnp.min(masked, axis=1, keepdims=True)
            n_it = (jnp.max(cnt) - float(k)).astype(jnp.int32)

            def fix(_, carry):
                t, r = carry
                live = r > 0.0
                t2 = jnp.min(jnp.where(arr > t, arr, jnp.inf), axis=1,
                             keepdims=True)
                t = jnp.where(live, t2, t)
                r = r - jnp.where(live, 1.0, 0.0)
                return t, r

            t, _ = jax.lax.fori_loop(0, n_it, fix, (t0, cnt - float(k)))
            return t

        cmw = cm.shape[1]
        if chunked and cmw % 1024 == 0:
            cm2 = jnp.max(cm.reshape(qbs, cmw // 1024, 8, 128),
                          axis=2).reshape(qbs, cmw // 8)
            c2 = kth_largest(cm2, m, top_k)
            c32 = refine(cm, c2, top_k)
        else:
            c32 = kth_largest(cm, m, top_k)
        t = refine(l, c32, top_k)
        scal[:, 1:2] = t

    @pl.when(ph == 1)
    def _():
        lb = logits[:, pl.ds(kb * kbs, kbs)]
        w = jnp.where(lb >= scal[:, 1:2], jnp.exp(lb - scal[:, 0:1]), 0.0)
        acc = jax.lax.dot_general(
            w.astype(jnp.bfloat16), v_ref[...], (((1,), (0,)), ((), ())),
            preferred_element_type=jnp.float32)
        dw = jnp.sum(w, axis=1, keepdims=True)

        @pl.when(kb == 0)
        def _():
            o_ref[...] = acc
            scal[:, 2:3] = dw

        @pl.when(kb > 0)
        def _():
            o_ref[...] = o_ref[...] + acc
            scal[:, 2:3] = scal[:, 2:3] + dw

    @pl.when(jnp.logical_and(ph == 1, kb == num_kb - 1))
    def _():
        o_ref[...] = o_ref[...] / scal[:, 2:3]


def _memory_read_2d(q2, kn2, v2, *, qbs, kbs):
    nq, ck = q2.shape
    nm, cv = v2.shape
    num_qb = nq // qbs
    num_kb = nm // kbs
    chunked = kbs % 1024 == 0
    cm_w = nm // 8 if chunked else nm

    body = functools.partial(_body, qbs=qbs, kbs=kbs, num_kb=num_kb, nm=nm,
                             top_k=_TOP_K, tau=_TAU, chunked=chunked)
    return pl.pallas_call(
        body,
        grid=(num_qb, 2, num_kb),
        in_specs=[
            pl.BlockSpec((qbs, ck), lambda qb, ph, kb: (qb, 0)),
            pl.BlockSpec((kbs, ck),
                         lambda qb, ph, kb: (jnp.where(ph == 0, kb, 0), 0)),
            pl.BlockSpec((kbs, cv),
                         lambda qb, ph, kb: (jnp.where(ph == 1, kb, 0), 0)),
        ],
        out_specs=pl.BlockSpec((qbs, cv), lambda qb, ph, kb: (qb, 0)),
        out_shape=jax.ShapeDtypeStruct((nq, cv), jnp.float32),
        scratch_shapes=[
            pltpu.VMEM((qbs, nm), jnp.float32),
            pltpu.VMEM((qbs, cm_w), jnp.float32),
            pltpu.VMEM((qbs, 3), jnp.float32),
        ],
    )(q2, kn2, v2)


def kernel(query_key, memory_keys, memory_values):
    b, ck, hq, wq = query_key.shape
    _, cv, hm, wm = memory_values.shape
    nq = hq * wq
    nm = hm * wm
    q2 = query_key.reshape(ck, nq).T
    k2 = memory_keys.reshape(ck, nm).T
    v2 = memory_values.reshape(cv, nm).T.astype(jnp.bfloat16)
    kn2 = _normalize_rows(k2, min(4096, nm))
    qbs = min(128, nq)
    kbs = min(8192, nm)
    read = _memory_read_2d(q2, kn2, v2, qbs=qbs, kbs=kbs)
    return read.T.reshape(b, cv, hq, wq)
